# baseline (device time: 12799 ns/iter reference)
import jax
import jax.numpy as jnp
from jax import lax
from jax.experimental import pallas as pl
from jax.experimental.pallas import tpu as pltpu

N_Y = 4


def kernel(x):
    m, n = x.shape
    c = m // N_Y

    def body(
        x_ref,
        out_ref,
        cast_ref,
        rs_ref,
        ag_src_ref,
        ag_ref,
        rs_send_sems,
        rs_recv_sems,
        ag_send_sems,
        ag_recv_sems,
    ):
        my_x = lax.axis_index("x")
        my_y = lax.axis_index("y")
        my_z = lax.axis_index("z")

        cast_ref[...] = x_ref[...].reshape(N_Y, c, n).astype(jnp.bfloat16)

        barrier_sem = pltpu.get_barrier_semaphore()
        for d in range(1, N_Y):
            peer = lax.rem(my_y + d, N_Y)
            pl.semaphore_signal(
                barrier_sem, inc=1,
                device_id=(my_x, peer, my_z),
                device_id_type=pl.DeviceIdType.MESH,
            )
        pl.semaphore_wait(barrier_sem, N_Y - 1)

        rs = []
        for d in range(1, N_Y):
            peer = lax.rem(my_y + d, N_Y)
            rdma = pltpu.make_async_remote_copy(
                src_ref=cast_ref.at[peer],
                dst_ref=rs_ref.at[N_Y - 1 - d],
                send_sem=rs_send_sems.at[d - 1],
                recv_sem=rs_recv_sems.at[N_Y - 1 - d],
                device_id=(my_x, peer, my_z),
                device_id_type=pl.DeviceIdType.MESH,
            )
            rdma.start()
            rs.append(rdma)
        for rdma in rs:
            rdma.wait()

        red = (
            x_ref[pl.ds(my_y * c, c), :]
            + rs_ref[0, :, :].astype(jnp.float32)
            + rs_ref[1, :, :].astype(jnp.float32)
            + rs_ref[2, :, :].astype(jnp.float32)
        )
        out_ref[pl.ds(my_y * c, c), :] = red
        ag_src_ref[...] = red.astype(jnp.bfloat16)

        ag = []
        for d in range(1, N_Y):
            peer = lax.rem(my_y + d, N_Y)
            rdma = pltpu.make_async_remote_copy(
                src_ref=ag_src_ref,
                dst_ref=ag_ref.at[N_Y - 1 - d],
                send_sem=ag_send_sems.at[d - 1],
                recv_sem=ag_recv_sems.at[N_Y - 1 - d],
                device_id=(my_x, peer, my_z),
                device_id_type=pl.DeviceIdType.MESH,
            )
            rdma.start()
            ag.append(rdma)
        for rdma in ag:
            rdma.wait()

        for j in range(N_Y - 1):
            owner = lax.rem(my_y + j + 1, N_Y)
            out_ref[pl.ds(owner * c, c), :] = ag_ref[j, :, :].astype(
                jnp.float32
            )

    return pl.pallas_call(
        body,
        out_shape=jax.ShapeDtypeStruct((m, n), x.dtype),
        in_specs=[pl.BlockSpec(memory_space=pltpu.VMEM)],
        out_specs=pl.BlockSpec(memory_space=pltpu.VMEM),
        scratch_shapes=[
            pltpu.VMEM((N_Y, c, n), jnp.bfloat16),
            pltpu.VMEM((N_Y - 1, c, n), jnp.bfloat16),
            pltpu.VMEM((c, n), jnp.bfloat16),
            pltpu.VMEM((N_Y - 1, c, n), jnp.bfloat16),
            pltpu.SemaphoreType.DMA((N_Y - 1,)),
            pltpu.SemaphoreType.DMA((N_Y - 1,)),
            pltpu.SemaphoreType.DMA((N_Y - 1,)),
            pltpu.SemaphoreType.DMA((N_Y - 1,)),
        ],
        compiler_params=pltpu.CompilerParams(collective_id=0),
    )(x)


# device time: 11454 ns/iter; 1.1174x vs baseline; 1.1174x over previous
import jax
import jax.numpy as jnp
from jax import lax
from jax.experimental import pallas as pl
from jax.experimental.pallas import tpu as pltpu

N_Y = 4


def kernel(x):
    m, n = x.shape

    def body(x_ref, out_ref, cast_ref, comm_ref, send_sems, recv_sems):
        my_x = lax.axis_index("x")
        my_y = lax.axis_index("y")
        my_z = lax.axis_index("z")

        barrier_sem = pltpu.get_barrier_semaphore()
        for d in range(1, N_Y):
            peer = lax.rem(my_y + d, N_Y)
            pl.semaphore_signal(
                barrier_sem, inc=1,
                device_id=(my_x, peer, my_z),
                device_id_type=pl.DeviceIdType.MESH,
            )

        cast_ref[...] = x_ref[...].astype(jnp.bfloat16)

        pl.semaphore_wait(barrier_sem, N_Y - 1)

        sends = []
        for d in range(1, N_Y):
            peer = lax.rem(my_y + d, N_Y)
            slot = N_Y - 1 - d
            rdma = pltpu.make_async_remote_copy(
                src_ref=cast_ref,
                dst_ref=comm_ref.at[slot],
                send_sem=send_sems.at[d - 1],
                recv_sem=recv_sems.at[slot],
                device_id=(my_x, peer, my_z),
                device_id_type=pl.DeviceIdType.MESH,
            )
            rdma.start()
            sends.append(rdma)

        for rdma in sends:
            rdma.wait()

        out_ref[...] = (
            x_ref[...]
            + comm_ref[0, :, :].astype(jnp.float32)
            + comm_ref[1, :, :].astype(jnp.float32)
            + comm_ref[2, :, :].astype(jnp.float32)
        ).astype(jnp.bfloat16)

    return pl.pallas_call(
        body,
        out_shape=jax.ShapeDtypeStruct((m, n), jnp.bfloat16),
        in_specs=[pl.BlockSpec(memory_space=pltpu.VMEM)],
        out_specs=pl.BlockSpec(memory_space=pltpu.VMEM),
        scratch_shapes=[
            pltpu.VMEM((m, n), jnp.bfloat16),
            pltpu.VMEM((N_Y - 1, m, n), jnp.bfloat16),
            pltpu.SemaphoreType.DMA((N_Y - 1,)),
            pltpu.SemaphoreType.DMA((N_Y - 1,)),
        ],
        compiler_params=pltpu.CompilerParams(collective_id=0),
    )(x)


# device time: 11330 ns/iter; 1.1297x vs baseline; 1.0109x over previous
import jax
import jax.numpy as jnp
from jax import lax
from jax.experimental import pallas as pl
from jax.experimental.pallas import tpu as pltpu

N_Y = 4


def kernel(x):
    m, n = x.shape

    def body(x_ref, out_ref, cast_ref, comm_ref, send_sems, recv_sems,
             ready_sems):
        my_x = lax.axis_index("x")
        my_y = lax.axis_index("y")
        my_z = lax.axis_index("z")

        barrier_sem = pltpu.get_barrier_semaphore()
        pl.semaphore_signal(barrier_sem, inc=1)
        pl.semaphore_wait(barrier_sem, 1)

        for d in range(1, N_Y):
            sender = lax.rem(my_y - d + N_Y, N_Y)
            pl.semaphore_signal(
                ready_sems.at[d - 1], inc=1,
                device_id=(my_x, sender, my_z),
                device_id_type=pl.DeviceIdType.MESH,
            )

        cast_ref[...] = x_ref[...].astype(jnp.bfloat16)

        sends = []
        for d in range(1, N_Y):
            peer = lax.rem(my_y + d, N_Y)
            slot = N_Y - 1 - d
            pl.semaphore_wait(ready_sems.at[d - 1], 1)
            rdma = pltpu.make_async_remote_copy(
                src_ref=cast_ref,
                dst_ref=comm_ref.at[slot],
                send_sem=send_sems.at[d - 1],
                recv_sem=recv_sems.at[slot],
                device_id=(my_x, peer, my_z),
                device_id_type=pl.DeviceIdType.MESH,
            )
            rdma.start()
            sends.append(rdma)

        for rdma in sends:
            rdma.wait()

        out_ref[...] = (
            x_ref[...]
            + comm_ref[0, :, :].astype(jnp.float32)
            + comm_ref[1, :, :].astype(jnp.float32)
            + comm_ref[2, :, :].astype(jnp.float32)
        ).astype(jnp.bfloat16)

    return pl.pallas_call(
        body,
        out_shape=jax.ShapeDtypeStruct((m, n), jnp.bfloat16),
        in_specs=[pl.BlockSpec(memory_space=pltpu.VMEM)],
        out_specs=pl.BlockSpec(memory_space=pltpu.VMEM),
        scratch_shapes=[
            pltpu.VMEM((m, n), jnp.bfloat16),
            pltpu.VMEM((N_Y - 1, m, n), jnp.bfloat16),
            pltpu.SemaphoreType.DMA((N_Y - 1,)),
            pltpu.SemaphoreType.DMA((N_Y - 1,)),
            pltpu.SemaphoreType.REGULAR((N_Y - 1,)),
        ],
        compiler_params=pltpu.CompilerParams(collective_id=0),
    )(x)


# device time: 11323 ns/iter; 1.1304x vs baseline; 1.0006x over previous
import jax
import jax.numpy as jnp
from jax import lax
from jax.experimental import pallas as pl
from jax.experimental.pallas import tpu as pltpu

N_Y = 4


def kernel(x):
    m, n = x.shape

    def body(x_ref, out_ref, cast_ref, comm_ref, send_sems, recv_sems,
             ready_sems):
        my_x = lax.axis_index("x")
        my_y = lax.axis_index("y")
        my_z = lax.axis_index("z")

        barrier_sem = pltpu.get_barrier_semaphore()
        pl.semaphore_signal(barrier_sem, inc=1)
        pl.semaphore_wait(barrier_sem, 1)

        for d in range(1, N_Y):
            sender = lax.rem(my_y - d + N_Y, N_Y)
            pl.semaphore_signal(
                ready_sems.at[d - 1], inc=1,
                device_id=(my_x, sender, my_z),
                device_id_type=pl.DeviceIdType.MESH,
            )

        cast_ref[...] = x_ref[...].astype(jnp.bfloat16)

        sends = []
        for d in range(1, N_Y):
            peer = lax.rem(my_y + d, N_Y)
            slot = N_Y - 1 - d
            pl.semaphore_wait(ready_sems.at[d - 1], 1)
            rdma = pltpu.make_async_remote_copy(
                src_ref=cast_ref,
                dst_ref=comm_ref.at[slot],
                send_sem=send_sems.at[d - 1],
                recv_sem=recv_sems.at[slot],
                device_id=(my_x, peer, my_z),
                device_id_type=pl.DeviceIdType.MESH,
            )
            rdma.start()
            sends.append(rdma)

        acc = x_ref[...]
        for j in range(N_Y - 1):
            sends[N_Y - 2 - j].wait_recv()
            acc = acc + comm_ref[j, :, :].astype(jnp.float32)
        out_ref[...] = acc.astype(jnp.bfloat16)

        for rdma in sends:
            rdma.wait_send()

    return pl.pallas_call(
        body,
        out_shape=jax.ShapeDtypeStruct((m, n), jnp.bfloat16),
        in_specs=[pl.BlockSpec(memory_space=pltpu.VMEM)],
        out_specs=pl.BlockSpec(memory_space=pltpu.VMEM),
        scratch_shapes=[
            pltpu.VMEM((m, n), jnp.bfloat16),
            pltpu.VMEM((N_Y - 1, m, n), jnp.bfloat16),
            pltpu.SemaphoreType.DMA((N_Y - 1,)),
            pltpu.SemaphoreType.DMA((N_Y - 1,)),
            pltpu.SemaphoreType.REGULAR((N_Y - 1,)),
        ],
        compiler_params=pltpu.CompilerParams(collective_id=0),
    )(x)
